# baseline (speedup 1.0000x reference)
import functools

import jax
import jax.numpy as jnp
from jax import lax
from jax.experimental import pallas as pl
from jax.experimental.pallas import tpu as pltpu
from jax.experimental.pallas import tpu_sc as plsc

N = 2048
E = 65536
STATE_DIM = 256
HID = 64
N_HEADS = 4

ROWS_PER_BLOCK = 512
BLOCK_WORDS = ROWS_PER_BLOCK * N
NUM_SUBCORES = 16
EDGES_PER_SUBCORE = E // NUM_SUBCORES
CHUNK = 128
NUM_CHUNKS = EDGES_PER_SUBCORE // CHUNK
ZERO_WORDS = 16384


def _build_adj(adj_row, adj_col, adj_val):
    mesh = plsc.VectorSubcoreMesh(core_axis_name="core", subcore_axis_name="subcore")

    @functools.partial(
        pl.kernel,
        out_type=jax.ShapeDtypeStruct((N, N), jnp.float32),
        mesh=mesh,
        scratch_types=[
            pltpu.VMEM((EDGES_PER_SUBCORE,), jnp.int32),
            pltpu.VMEM((EDGES_PER_SUBCORE,), jnp.int32),
            pltpu.VMEM((EDGES_PER_SUBCORE,), jnp.float32),
            pltpu.VMEM((2, NUM_CHUNKS, CHUNK), jnp.int32),
            pltpu.VMEM((ZERO_WORDS,), jnp.float32),
            pltpu.VMEM_SHARED((BLOCK_WORDS + 16,), jnp.float32),
            pltpu.SemaphoreType.DMA,
        ],
    )
    def build(row_hbm, col_hbm, val_hbm, out_hbm, row_v, col_v, val_v, idx_v,
              zero_v, acc_sh, sem):
        c = lax.axis_index("core")
        s = lax.axis_index("subcore")

        e0 = s * EDGES_PER_SUBCORE
        d_row = pltpu.async_copy(row_hbm.at[pl.ds(e0, EDGES_PER_SUBCORE)],
                                 row_v, sem)
        d_col = pltpu.async_copy(col_hbm.at[pl.ds(e0, EDGES_PER_SUBCORE)],
                                 col_v, sem)
        d_val = pltpu.async_copy(val_hbm.at[pl.ds(e0, EDGES_PER_SUBCORE)],
                                 val_v, sem)

        @pl.loop(0, ZERO_WORDS, step=16)
        def _(i):
            zero_v[pl.ds(i, 16)] = jnp.zeros((16,), jnp.float32)

        d_row.wait()
        d_col.wait()
        d_val.wait()

        words_per_sub = BLOCK_WORDS // NUM_SUBCORES
        rows_per_sub = ROWS_PER_BLOCK // NUM_SUBCORES

        zdescs = [
            pltpu.async_copy(
                zero_v,
                acc_sh.at[pl.ds(s * words_per_sub + z * ZERO_WORDS,
                                ZERO_WORDS)], sem)
            for z in range(words_per_sub // ZERO_WORDS)]

        @pl.when(s == 0)
        def _():
            pltpu.sync_copy(zero_v.at[pl.ds(0, 16)],
                            acc_sh.at[pl.ds(BLOCK_WORDS, 16)])

        for p in range(2):
            row_lo0 = (2 * c + p) * ROWS_PER_BLOCK

            @pl.loop(0, NUM_CHUNKS)
            def _(k, row_lo0=row_lo0, p=p):
                @pl.loop(0, CHUNK, step=16)
                def _(j, k=k):
                    off = k * CHUNK + j
                    r = row_v[pl.ds(off, 16)]
                    cc = col_v[pl.ds(off, 16)]
                    rb = r - row_lo0
                    ok = (rb >= 0) & (rb < ROWS_PER_BLOCK)
                    flat = jnp.where(ok, rb * N + cc, BLOCK_WORDS + s)
                    idx_v[p, k, pl.ds(j, 16)] = flat

        for p in range(2):
            block = 2 * c + p

            for d in zdescs:
                d.wait()
            plsc.subcore_barrier()

            sdescs = [
                pltpu.async_copy(val_v.at[pl.ds(k * CHUNK, CHUNK)],
                                 acc_sh.at[idx_v.at[p, k]], sem, add=True)
                for k in range(NUM_CHUNKS)]
            for d in sdescs:
                d.wait()

            plsc.subcore_barrier()

            wdescs = [
                pltpu.async_copy(
                    acc_sh.at[pl.ds(s * words_per_sub + r * N, N)],
                    out_hbm.at[block * ROWS_PER_BLOCK + s * rows_per_sub + r],
                    sem)
                for r in range(rows_per_sub)]
            for d in wdescs:
                d.wait()

            if p == 0:
                zdescs = [
                    pltpu.async_copy(
                        zero_v,
                        acc_sh.at[pl.ds(s * words_per_sub + z * ZERO_WORDS,
                                        ZERO_WORDS)], sem)
                    for z in range(words_per_sub // ZERO_WORDS)]
            else:
                zdescs = []
            plsc.subcore_barrier()

    return build(adj_row, adj_col, adj_val)


def _mm_kernel(x_ref, w_ref, o_ref):
    o_ref[...] = jnp.dot(x_ref[...], w_ref[...],
                         preferred_element_type=jnp.float32)


def _input_proj(state, gc1_w):
    return pl.pallas_call(
        _mm_kernel,
        out_shape=jax.ShapeDtypeStruct((N, HID), jnp.float32),
    )(state, gc1_w)


def _layer1_kernel(a_ref, y_ref, b1_ref, w2_ref, o_ref):
    s1 = jnp.dot(a_ref[...], y_ref[...], preferred_element_type=jnp.float32,
                 precision=lax.Precision.DEFAULT)
    x1 = jnp.maximum(s1 + b1_ref[...], 0.0)
    o_ref[...] = jnp.dot(x1, w2_ref[...], preferred_element_type=jnp.float32)


def _layer1(A, y1, gc1_b, gc2_w):
    blk = 512
    return pl.pallas_call(
        _layer1_kernel,
        grid=(N // blk,),
        in_specs=[
            pl.BlockSpec((blk, N), lambda i: (i, 0)),
            pl.BlockSpec((N, HID), lambda i: (0, 0)),
            pl.BlockSpec((1, HID), lambda i: (0, 0)),
            pl.BlockSpec((HID, HID), lambda i: (0, 0)),
        ],
        out_specs=pl.BlockSpec((blk, HID), lambda i: (i, 0)),
        out_shape=jax.ShapeDtypeStruct((N, HID), jnp.float32),
    )(A, y1, gc1_b, gc2_w)


def _layer2_kernel(a_ref, y_ref, b2_ref, o_ref):
    s2 = jnp.dot(a_ref[...], y_ref[...], preferred_element_type=jnp.float32,
                 precision=lax.Precision.DEFAULT)
    o_ref[...] = jnp.maximum(s2 + b2_ref[...], 0.0)


def _layer2(A, y2, gc2_b):
    blk = 512
    return pl.pallas_call(
        _layer2_kernel,
        grid=(N // blk,),
        in_specs=[
            pl.BlockSpec((blk, N), lambda i: (i, 0)),
            pl.BlockSpec((N, HID), lambda i: (0, 0)),
            pl.BlockSpec((1, HID), lambda i: (0, 0)),
        ],
        out_specs=pl.BlockSpec((blk, HID), lambda i: (i, 0)),
        out_shape=jax.ShapeDtypeStruct((N, HID), jnp.float32),
    )(A, y2, gc2_b)


def _gat_mlp_kernel(x2_ref, x2b_ref, w3_ref, asrc_ref, adst_ref,
                    f1w_ref, f1b_ref, f2w_ref, f2b_ref, o_ref):
    x2f = x2_ref[...]
    x2b = x2b_ref[...]
    w3 = w3_ref[...]
    asrc = asrc_ref[...]
    adst = adst_ref[...]

    dn = (((1,), (1,)), ((), ()))
    acc = jnp.zeros((x2b.shape[0], HID), jnp.float32)
    for h in range(N_HEADS):
        hh = jnp.dot(x2f, w3[h], preferred_element_type=jnp.float32,
                     precision=lax.Precision.DEFAULT)
        hb = jnp.dot(x2b, w3[h], preferred_element_type=jnp.float32,
                     precision=lax.Precision.DEFAULT)
        ed = lax.dot_general(adst[h:h + 1, :], hh, dn,
                             preferred_element_type=jnp.float32)
        es = lax.dot_general(hb, asrc[h:h + 1, :], dn,
                             preferred_element_type=jnp.float32)
        t = es + jnp.max(ed, axis=1, keepdims=True)
        m = jnp.maximum(t, 0.01 * t)
        u = (es - m) + ed
        v = (0.01 * es - m) + 0.01 * ed
        p = jnp.exp(jnp.maximum(u, v))
        den = jnp.sum(p, axis=1, keepdims=True)
        oh = jnp.dot(p, hh, preferred_element_type=jnp.float32,
                     precision=lax.Precision.DEFAULT)
        acc = acc + oh / den
    att = acc * (1.0 / N_HEADS)

    xc = jnp.concatenate([x2b, att], axis=1)
    hdn = jnp.maximum(
        jnp.dot(xc, f1w_ref[...], preferred_element_type=jnp.float32)
        + f1b_ref[...], 0.0)
    lg = (jnp.dot(hdn, f2w_ref[...], preferred_element_type=jnp.float32)
          + f2b_ref[...])
    m2 = jnp.max(lg, axis=1, keepdims=True)
    e2 = jnp.exp(lg - m2)
    o_ref[...] = e2 / jnp.sum(e2, axis=1, keepdims=True)


def _gat_mlp(x2, att_W, a_src, a_dst, fc1_w, fc1_b, fc2_w, fc2_b):
    blk = 512
    return pl.pallas_call(
        _gat_mlp_kernel,
        grid=(N // blk,),
        in_specs=[
            pl.BlockSpec((N, HID), lambda i: (0, 0)),
            pl.BlockSpec((blk, HID), lambda i: (i, 0)),
            pl.BlockSpec((N_HEADS, HID, HID), lambda i: (0, 0, 0)),
            pl.BlockSpec((N_HEADS, HID), lambda i: (0, 0)),
            pl.BlockSpec((N_HEADS, HID), lambda i: (0, 0)),
            pl.BlockSpec((2 * HID, HID), lambda i: (0, 0)),
            pl.BlockSpec((1, HID), lambda i: (0, 0)),
            pl.BlockSpec((HID, HID), lambda i: (0, 0)),
            pl.BlockSpec((1, HID), lambda i: (0, 0)),
        ],
        out_specs=pl.BlockSpec((blk, HID), lambda i: (i, 0)),
        out_shape=jax.ShapeDtypeStruct((N, HID), jnp.float32),
    )(x2, x2, att_W, a_src, a_dst, fc1_w, fc1_b, fc2_w, fc2_b)


def kernel(state, adj_row, adj_col, adj_val, gc1_w, gc1_b, gc2_w, gc2_b,
           att_W, att_a, fc1_w, fc1_b, fc2_w, fc2_b):
    adj_row = adj_row.astype(jnp.int32)
    adj_col = adj_col.astype(jnp.int32)

    A = _build_adj(adj_row, adj_col, adj_val)

    y1 = _input_proj(state, gc1_w)
    y2 = _layer1(A, y1, gc1_b.reshape(1, HID), gc2_w)
    x2 = _layer2(A, y2, gc2_b.reshape(1, HID))

    a_src = att_a[:, :HID, 0]
    a_dst = att_a[:, HID:, 0]
    return _gat_mlp(x2, att_W, a_src, a_dst, fc1_w, fc1_b.reshape(1, HID),
                    fc2_w, fc2_b.reshape(1, HID))


# speedup vs baseline: 1.0062x; 1.0062x over previous
import functools

import jax
import jax.numpy as jnp
from jax import lax
from jax.experimental import pallas as pl
from jax.experimental.pallas import tpu as pltpu
from jax.experimental.pallas import tpu_sc as plsc

N = 2048
E = 65536
STATE_DIM = 256
HID = 64
N_HEADS = 4

ROWS_PER_BLOCK = 512
BLOCK_WORDS = ROWS_PER_BLOCK * N
NUM_SUBCORES = 16
EDGES_PER_SUBCORE = E // NUM_SUBCORES
CHUNK = 128
NUM_CHUNKS = EDGES_PER_SUBCORE // CHUNK
ZERO_WORDS = 16384


def _build_adj(adj_row, adj_col, adj_val):
    mesh = plsc.VectorSubcoreMesh(core_axis_name="core", subcore_axis_name="subcore")

    @functools.partial(
        pl.kernel,
        out_type=jax.ShapeDtypeStruct((N, N), jnp.float32),
        mesh=mesh,
        scratch_types=[
            pltpu.VMEM((EDGES_PER_SUBCORE,), jnp.int32),
            pltpu.VMEM((EDGES_PER_SUBCORE,), jnp.int32),
            pltpu.VMEM((EDGES_PER_SUBCORE,), jnp.float32),
            pltpu.VMEM((2, NUM_CHUNKS, CHUNK), jnp.int32),
            pltpu.VMEM((ZERO_WORDS,), jnp.float32),
            pltpu.VMEM_SHARED((BLOCK_WORDS + 16,), jnp.float32),
            pltpu.SemaphoreType.DMA,
        ],
    )
    def build(row_hbm, col_hbm, val_hbm, out_hbm, row_v, col_v, val_v, idx_v,
              zero_v, acc_sh, sem):
        c = lax.axis_index("core")
        s = lax.axis_index("subcore")

        e0 = s * EDGES_PER_SUBCORE
        d_row = pltpu.async_copy(row_hbm.at[pl.ds(e0, EDGES_PER_SUBCORE)],
                                 row_v, sem)
        d_col = pltpu.async_copy(col_hbm.at[pl.ds(e0, EDGES_PER_SUBCORE)],
                                 col_v, sem)
        d_val = pltpu.async_copy(val_hbm.at[pl.ds(e0, EDGES_PER_SUBCORE)],
                                 val_v, sem)

        @pl.loop(0, ZERO_WORDS, step=16)
        def _(i):
            zero_v[pl.ds(i, 16)] = jnp.zeros((16,), jnp.float32)

        d_row.wait()
        d_col.wait()
        d_val.wait()

        words_per_sub = BLOCK_WORDS // NUM_SUBCORES
        rows_per_sub = ROWS_PER_BLOCK // NUM_SUBCORES

        zdescs = [
            pltpu.async_copy(
                zero_v,
                acc_sh.at[pl.ds(s * words_per_sub + z * ZERO_WORDS,
                                ZERO_WORDS)], sem)
            for z in range(words_per_sub // ZERO_WORDS)]

        @pl.when(s == 0)
        def _():
            pltpu.sync_copy(zero_v.at[pl.ds(0, 16)],
                            acc_sh.at[pl.ds(BLOCK_WORDS, 16)])

        for p in range(2):
            row_lo0 = (2 * c + p) * ROWS_PER_BLOCK

            @pl.loop(0, NUM_CHUNKS)
            def _(k, row_lo0=row_lo0, p=p):
                @pl.loop(0, CHUNK, step=16)
                def _(j, k=k):
                    off = k * CHUNK + j
                    r = row_v[pl.ds(off, 16)]
                    cc = col_v[pl.ds(off, 16)]
                    rb = r - row_lo0
                    ok = (rb >= 0) & (rb < ROWS_PER_BLOCK)
                    flat = jnp.where(ok, rb * N + cc, BLOCK_WORDS + s)
                    idx_v[p, k, pl.ds(j, 16)] = flat

        for p in range(2):
            block = 2 * c + p

            for d in zdescs:
                d.wait()
            plsc.subcore_barrier()

            sdescs = [
                pltpu.async_copy(val_v.at[pl.ds(k * CHUNK, CHUNK)],
                                 acc_sh.at[idx_v.at[p, k]], sem, add=True)
                for k in range(NUM_CHUNKS)]
            for d in sdescs:
                d.wait()

            plsc.subcore_barrier()

            wdescs = [
                pltpu.async_copy(
                    acc_sh.at[pl.ds(s * words_per_sub + r * N, N)],
                    out_hbm.at[block * ROWS_PER_BLOCK + s * rows_per_sub + r],
                    sem)
                for r in range(rows_per_sub)]
            for d in wdescs:
                d.wait()

            if p == 0:
                zdescs = [
                    pltpu.async_copy(
                        zero_v,
                        acc_sh.at[pl.ds(s * words_per_sub + z * ZERO_WORDS,
                                        ZERO_WORDS)], sem)
                    for z in range(words_per_sub // ZERO_WORDS)]
            else:
                zdescs = []
            plsc.subcore_barrier()

    return build(adj_row, adj_col, adj_val)


def _mm_kernel(x_ref, w_ref, o_ref):
    o_ref[...] = jnp.dot(x_ref[...], w_ref[...],
                         preferred_element_type=jnp.float32)


def _input_proj(state, gc1_w):
    return pl.pallas_call(
        _mm_kernel,
        out_shape=jax.ShapeDtypeStruct((N, HID), jnp.float32),
    )(state, gc1_w)


def _layer1_kernel(a_ref, y_ref, b1_ref, w2_ref, o_ref):
    s1 = jnp.dot(a_ref[...], y_ref[...], preferred_element_type=jnp.float32)
    x1 = jnp.maximum(s1 + b1_ref[...], 0.0)
    o_ref[...] = jnp.dot(x1, w2_ref[...], preferred_element_type=jnp.float32)


def _layer1(A, y1, gc1_b, gc2_w):
    blk = 512
    return pl.pallas_call(
        _layer1_kernel,
        grid=(N // blk,),
        in_specs=[
            pl.BlockSpec((blk, N), lambda i: (i, 0)),
            pl.BlockSpec((N, HID), lambda i: (0, 0)),
            pl.BlockSpec((1, HID), lambda i: (0, 0)),
            pl.BlockSpec((HID, HID), lambda i: (0, 0)),
        ],
        out_specs=pl.BlockSpec((blk, HID), lambda i: (i, 0)),
        out_shape=jax.ShapeDtypeStruct((N, HID), jnp.float32),
    )(A, y1, gc1_b, gc2_w)


def _layer2_kernel(a_ref, y_ref, b2_ref, o_ref):
    s2 = jnp.dot(a_ref[...], y_ref[...], preferred_element_type=jnp.float32)
    o_ref[...] = jnp.maximum(s2 + b2_ref[...], 0.0)


def _layer2(A, y2, gc2_b):
    blk = 512
    return pl.pallas_call(
        _layer2_kernel,
        grid=(N // blk,),
        in_specs=[
            pl.BlockSpec((blk, N), lambda i: (i, 0)),
            pl.BlockSpec((N, HID), lambda i: (0, 0)),
            pl.BlockSpec((1, HID), lambda i: (0, 0)),
        ],
        out_specs=pl.BlockSpec((blk, HID), lambda i: (i, 0)),
        out_shape=jax.ShapeDtypeStruct((N, HID), jnp.float32),
    )(A, y2, gc2_b)


def _gat_mlp_kernel(x2_ref, w3_ref, aa_ref, f1w_ref, f1b_ref, f2w_ref,
                    f2b_ref, o_ref, *, blk):
    i = pl.program_id(0)
    x2f = x2_ref[...]
    x2b = x2_ref[pl.ds(i * blk, blk), :]
    w3 = w3_ref[...]
    aa = aa_ref[...]
    asrc = aa[:, :HID]
    adst = aa[:, HID:]

    dn = (((1,), (1,)), ((), ()))
    acc = jnp.zeros((x2b.shape[0], HID), jnp.float32)
    for h in range(N_HEADS):
        hh = jnp.dot(x2f, w3[h], preferred_element_type=jnp.float32,
                     precision=lax.Precision.DEFAULT)
        hb = jnp.dot(x2b, w3[h], preferred_element_type=jnp.float32,
                     precision=lax.Precision.DEFAULT)
        ed = lax.dot_general(adst[h:h + 1, :], hh, dn,
                             preferred_element_type=jnp.float32)
        es = lax.dot_general(hb, asrc[h:h + 1, :], dn,
                             preferred_element_type=jnp.float32)
        t = es + jnp.max(ed, axis=1, keepdims=True)
        m = jnp.maximum(t, 0.01 * t)
        u = (es - m) + ed
        v = (0.01 * es - m) + 0.01 * ed
        p = jnp.exp(jnp.maximum(u, v))
        den = jnp.sum(p, axis=1, keepdims=True)
        oh = jnp.dot(p, hh, preferred_element_type=jnp.float32,
                     precision=lax.Precision.DEFAULT)
        acc = acc + oh / den
    att = acc * (1.0 / N_HEADS)

    xc = jnp.concatenate([x2b, att], axis=1)
    hdn = jnp.maximum(
        jnp.dot(xc, f1w_ref[...], preferred_element_type=jnp.float32)
        + f1b_ref[...], 0.0)
    lg = (jnp.dot(hdn, f2w_ref[...], preferred_element_type=jnp.float32)
          + f2b_ref[...])
    m2 = jnp.max(lg, axis=1, keepdims=True)
    e2 = jnp.exp(lg - m2)
    o_ref[...] = e2 / jnp.sum(e2, axis=1, keepdims=True)


def _gat_mlp(x2, att_W, att_a2, fc1_w, fc1_b, fc2_w, fc2_b):
    blk = 512
    return pl.pallas_call(
        functools.partial(_gat_mlp_kernel, blk=blk),
        grid=(N // blk,),
        in_specs=[
            pl.BlockSpec((N, HID), lambda i: (0, 0)),
            pl.BlockSpec((N_HEADS, HID, HID), lambda i: (0, 0, 0)),
            pl.BlockSpec((N_HEADS, 2 * HID), lambda i: (0, 0)),
            pl.BlockSpec((2 * HID, HID), lambda i: (0, 0)),
            pl.BlockSpec((1, HID), lambda i: (0, 0)),
            pl.BlockSpec((HID, HID), lambda i: (0, 0)),
            pl.BlockSpec((1, HID), lambda i: (0, 0)),
        ],
        out_specs=pl.BlockSpec((blk, HID), lambda i: (i, 0)),
        out_shape=jax.ShapeDtypeStruct((N, HID), jnp.float32),
    )(x2, att_W, att_a2, fc1_w, fc1_b, fc2_w, fc2_b)


def kernel(state, adj_row, adj_col, adj_val, gc1_w, gc1_b, gc2_w, gc2_b,
           att_W, att_a, fc1_w, fc1_b, fc2_w, fc2_b):
    adj_row = adj_row.astype(jnp.int32)
    adj_col = adj_col.astype(jnp.int32)

    A = _build_adj(adj_row, adj_col, adj_val)

    y1 = _input_proj(state, gc1_w)
    y2 = _layer1(A, y1, gc1_b.reshape(1, HID), gc2_w)
    x2 = _layer2(A, y2, gc2_b.reshape(1, HID))

    return _gat_mlp(x2, att_W, att_a.reshape(N_HEADS, 2 * HID), fc1_w,
                    fc1_b.reshape(1, HID), fc2_w, fc2_b.reshape(1, HID))
